# Initial kernel scaffold; baseline (speedup 1.0000x reference)
import jax
import jax.numpy as jnp
from jax.experimental import pallas as pl


def kernel(X, adj_t, tuples_coo):
    raise NotImplementedError("write your pallas kernel here")


# speedup vs baseline: 4.8228x; 4.8228x over previous
import functools

import jax
import jax.numpy as jnp
from jax import lax
from jax.experimental import pallas as pl
from jax.experimental.pallas import tpu as pltpu
from jax.experimental.pallas import tpu_sc as plsc

_B = 320000
_D = 128
_NC, _NS = 2, 16
_NW = _NC * _NS
_TPW = _B // _NW
_G = 80
_NCH = _TPW // _G
_LANES = 16


def _make_sc_kernel():
    mesh = plsc.VectorSubcoreMesh(core_axis_name="c", subcore_axis_name="s")

    @functools.partial(
        pl.kernel,
        mesh=mesh,
        out_type=jax.ShapeDtypeStruct((_B, _D), jnp.float32),
        scratch_types=[
            pltpu.VMEM((_TPW,), jnp.int32),
            pltpu.VMEM((_TPW,), jnp.int32),
            pltpu.VMEM((_TPW,), jnp.int32),
            pltpu.VMEM((_G, _D), jnp.float32),
            pltpu.VMEM((_G, _D), jnp.float32),
            pltpu.VMEM((_G, _D), jnp.float32),
            pltpu.SemaphoreType.DMA,
            pltpu.SemaphoreType.DMA,
            pltpu.SemaphoreType.DMA,
        ],
    )
    def k(x_hbm, idx_hbm, out_hbm, idx0, idx1, idx2, r0, r1, r2, s0, s1, s2):
        wid = lax.axis_index("s") * _NC + lax.axis_index("c")
        base = wid * _TPW
        pltpu.sync_copy(idx_hbm.at[pl.ds(base, _TPW)], idx0)
        pltpu.sync_copy(idx_hbm.at[pl.ds(_B + base, _TPW)], idx1)
        pltpu.sync_copy(idx_hbm.at[pl.ds(2 * _B + base, _TPW)], idx2)

        def chunk(c, carry):
            off = pl.multiple_of(c * _G, 8)
            c0 = pltpu.async_copy(x_hbm.at[idx0.at[pl.ds(off, _G)]], r0, s0)
            c1 = pltpu.async_copy(x_hbm.at[idx1.at[pl.ds(off, _G)]], r1, s1)
            c2 = pltpu.async_copy(x_hbm.at[idx2.at[pl.ds(off, _G)]], r2, s2)
            c0.wait()
            c1.wait()
            c2.wait()

            def row(rr, acc):
                for j in range(_D // _LANES):
                    sl = pl.ds(j * _LANES, _LANES)
                    r0[rr, sl] = r0[rr, sl] * r1[rr, sl] * r2[rr, sl]
                return acc

            lax.fori_loop(0, _G, row, 0)
            pltpu.sync_copy(r0, out_hbm.at[pl.ds(base + off, _G), :])
            return carry

        lax.fori_loop(0, _NCH, chunk, 0)

    return k


_sc_prod = _make_sc_kernel()


def kernel(X, adj_t, tuples_coo):
    del adj_t
    idx = tuples_coo.astype(jnp.int32).reshape(-1)
    return _sc_prod(X, idx)


# speedup vs baseline: 7.2939x; 1.5124x over previous
import functools

import jax
import jax.numpy as jnp
from jax import lax
from jax.experimental import pallas as pl
from jax.experimental.pallas import tpu as pltpu
from jax.experimental.pallas import tpu_sc as plsc

_B = 320000
_D = 128
_NC, _NS = 2, 16
_NW = _NC * _NS
_TPW = _B // _NW
_G = 40
_NCH = _TPW // _G
_NP = _NCH // 2
_LANES = 16


def _make_sc_kernel():
    mesh = plsc.VectorSubcoreMesh(core_axis_name="c", subcore_axis_name="s")

    @functools.partial(
        pl.kernel,
        mesh=mesh,
        out_type=jax.ShapeDtypeStruct((_B, _D), jnp.float32),
        scratch_types=[
            pltpu.VMEM((_TPW,), jnp.int32),
            pltpu.VMEM((_TPW,), jnp.int32),
            pltpu.VMEM((_TPW,), jnp.int32),
            pltpu.VMEM((_G, _D), jnp.float32),
            pltpu.VMEM((_G, _D), jnp.float32),
            pltpu.VMEM((_G, _D), jnp.float32),
            pltpu.VMEM((_G, _D), jnp.float32),
            pltpu.VMEM((_G, _D), jnp.float32),
            pltpu.VMEM((_G, _D), jnp.float32),
            pltpu.VMEM((_G, _D), jnp.float32),
            pltpu.VMEM((_G, _D), jnp.float32),
            pltpu.SemaphoreType.DMA,
            pltpu.SemaphoreType.DMA,
            pltpu.SemaphoreType.DMA,
            pltpu.SemaphoreType.DMA,
        ],
    )
    def k(x_hbm, idx_hbm, out_hbm, idx0, idx1, idx2,
          r0a, r1a, r2a, r0b, r1b, r2b, oa, ob, sga, sgb, soa, sob):
        wid = lax.axis_index("s") * _NC + lax.axis_index("c")
        base = wid * _TPW
        pltpu.sync_copy(idx_hbm.at[pl.ds(base, _TPW)], idx0)
        pltpu.sync_copy(idx_hbm.at[pl.ds(_B + base, _TPW)], idx1)
        pltpu.sync_copy(idx_hbm.at[pl.ds(2 * _B + base, _TPW)], idx2)

        idxs = (idx0, idx1, idx2)
        set_a = (r0a, r1a, r2a)
        set_b = (r0b, r1b, r2b)

        def start_g(rs, sem, off):
            for iv, rv in zip(idxs, rs):
                pltpu.async_copy(x_hbm.at[iv.at[pl.ds(off, _G)]], rv, sem)

        def wait_g(rs, sem):
            for rv in rs:
                pltpu.make_async_copy(x_hbm.at[pl.ds(0, _G)], rv, sem).wait()

        def start_out(ov, sem, off):
            pltpu.async_copy(ov, out_hbm.at[pl.ds(base + off, _G), :], sem)

        def wait_out(ov, sem):
            pltpu.make_async_copy(ov, out_hbm.at[pl.ds(base, _G), :], sem).wait()

        def compute(rs, ov):
            r0v, r1v, r2v = rs

            def row(rr, acc):
                for j in range(_D // _LANES):
                    sl = pl.ds(j * _LANES, _LANES)
                    ov[rr, sl] = r0v[rr, sl] * r1v[rr, sl] * r2v[rr, sl]
                return acc

            lax.fori_loop(0, _G, row, 0)

        start_g(set_a, sga, 0)

        def pair(p, carry):
            off0 = pl.multiple_of(2 * p * _G, 8)
            off1 = pl.multiple_of((2 * p + 1) * _G, 8)
            start_g(set_b, sgb, off1)
            wait_g(set_a, sga)

            @pl.when(p > 0)
            def _():
                wait_out(oa, soa)

            compute(set_a, oa)
            start_out(oa, soa, off0)

            @pl.when(p < _NP - 1)
            def _():
                start_g(set_a, sga, pl.multiple_of((2 * p + 2) * _G, 8))

            wait_g(set_b, sgb)

            @pl.when(p > 0)
            def _():
                wait_out(ob, sob)

            compute(set_b, ob)
            start_out(ob, sob, off1)
            return carry

        lax.fori_loop(0, _NP, pair, 0)
        wait_out(oa, soa)
        wait_out(ob, sob)

    return k


_sc_prod = _make_sc_kernel()


def kernel(X, adj_t, tuples_coo):
    del adj_t
    idx = tuples_coo.astype(jnp.int32).reshape(-1)
    return _sc_prod(X, idx)
